# baseline (speedup 1.0000x reference)
import jax
import jax.numpy as jnp
from jax.experimental import pallas as pl
from jax.experimental.pallas import tpu as pltpu

_HIDDEN = 2048
_INTER = 2048
_TOKENS = 32
_BK = 256
_NA = _HIDDEN // _BK
_NB = _INTER // _BK


def _mlp_kernel(x_ref, wg_ref, wu_ref, wd_ref, o_ref, g_ref, u_ref):
    i = pl.program_id(0)

    @pl.when(i < _NA)
    def _phase_a():
        xb = x_ref[:, pl.ds(i * _BK, _BK)]
        g = jnp.dot(xb, wg_ref[...], preferred_element_type=jnp.float32)
        u = jnp.dot(xb, wu_ref[...], preferred_element_type=jnp.float32)

        @pl.when(i == 0)
        def _init():
            g_ref[...] = g
            u_ref[...] = u

        @pl.when(i > 0)
        def _acc():
            g_ref[...] += g
            u_ref[...] += u

    @pl.when(i >= _NA)
    def _phase_b():
        j = i - _NA
        gb = g_ref[:, pl.ds(j * _BK, _BK)]
        ub = u_ref[:, pl.ds(j * _BK, _BK)]
        act = gb * jax.nn.sigmoid(gb) * ub
        contrib = jnp.dot(act, wd_ref[...], preferred_element_type=jnp.float32)

        @pl.when(j == 0)
        def _init():
            o_ref[...] = contrib

        @pl.when(j > 0)
        def _acc():
            o_ref[...] += contrib


def kernel(x, W_gate, W_up, W_down):
    return pl.pallas_call(
        _mlp_kernel,
        grid=(_NA + _NB,),
        in_specs=[
            pl.BlockSpec((_TOKENS, _HIDDEN), lambda i: (0, 0)),
            pl.BlockSpec((_BK, _INTER), lambda i: (jnp.minimum(i, _NA - 1), 0)),
            pl.BlockSpec((_BK, _INTER), lambda i: (jnp.minimum(i, _NA - 1), 0)),
            pl.BlockSpec((_BK, _HIDDEN), lambda i: (jnp.maximum(i - _NA, 0), 0)),
        ],
        out_specs=pl.BlockSpec((_TOKENS, _HIDDEN), lambda i: (0, 0)),
        out_shape=jax.ShapeDtypeStruct((_TOKENS, _HIDDEN), jnp.float32),
        scratch_shapes=[
            pltpu.VMEM((_TOKENS, _INTER), jnp.float32),
            pltpu.VMEM((_TOKENS, _INTER), jnp.float32),
        ],
    )(x, W_gate, W_up, W_down)


# speedup vs baseline: 1.1436x; 1.1436x over previous
import jax
import jax.numpy as jnp
from jax.experimental import pallas as pl

_HIDDEN = 2048
_INTER = 2048
_TOKENS = 32
_BJ = 256


def _split_dot(a, w):
    w16 = w.astype(jnp.bfloat16)
    a_hi = a.astype(jnp.bfloat16)
    a_lo = (a - a_hi.astype(jnp.float32)).astype(jnp.bfloat16)
    return (jnp.dot(a_hi, w16, preferred_element_type=jnp.float32)
            + jnp.dot(a_lo, w16, preferred_element_type=jnp.float32))


def _mlp_kernel(x_ref, wg_ref, wu_ref, wd_ref, o_ref):
    j = pl.program_id(0)
    x = x_ref[...]
    gate = _split_dot(x, wg_ref[...])
    up = _split_dot(x, wu_ref[...])
    act = gate * jax.nn.sigmoid(gate) * up
    contrib = _split_dot(act, wd_ref[...])

    @pl.when(j == 0)
    def _init():
        o_ref[...] = contrib

    @pl.when(j > 0)
    def _acc():
        o_ref[...] += contrib


def kernel(x, W_gate, W_up, W_down):
    return pl.pallas_call(
        _mlp_kernel,
        grid=(_INTER // _BJ,),
        in_specs=[
            pl.BlockSpec((_TOKENS, _HIDDEN), lambda j: (0, 0)),
            pl.BlockSpec((_HIDDEN, _BJ), lambda j: (0, j)),
            pl.BlockSpec((_HIDDEN, _BJ), lambda j: (0, j)),
            pl.BlockSpec((_BJ, _HIDDEN), lambda j: (j, 0)),
        ],
        out_specs=pl.BlockSpec((_TOKENS, _HIDDEN), lambda j: (0, 0)),
        out_shape=jax.ShapeDtypeStruct((_TOKENS, _HIDDEN), jnp.float32),
    )(x, W_gate, W_up, W_down)


# speedup vs baseline: 1.2210x; 1.0677x over previous
import jax
import jax.numpy as jnp
from jax.experimental import pallas as pl
from jax.experimental.pallas import tpu as pltpu

_HIDDEN = 2048
_INTER = 2048
_TOKENS = 32
_BJ = 256
_NJ = _INTER // _BJ


def _mlp_kernel(x_ref, wg_ref, wu_ref, wd_ref, o_ref, act_ref):
    j = pl.program_id(0)

    @pl.when(j > 0)
    def _down_prev():
        contrib = jnp.dot(act_ref[...], wd_ref[...],
                          preferred_element_type=jnp.float32)

        @pl.when(j == 1)
        def _init():
            o_ref[...] = contrib

        @pl.when(j > 1)
        def _acc():
            o_ref[...] += contrib

    @pl.when(j < _NJ)
    def _gate_up():
        x = x_ref[...]
        gate = jnp.dot(x, wg_ref[...], preferred_element_type=jnp.float32)
        up = jnp.dot(x, wu_ref[...], preferred_element_type=jnp.float32)
        act_ref[...] = gate * jax.nn.sigmoid(gate) * up


def kernel(x, W_gate, W_up, W_down):
    return pl.pallas_call(
        _mlp_kernel,
        grid=(_NJ + 1,),
        in_specs=[
            pl.BlockSpec((_TOKENS, _HIDDEN), lambda j: (0, 0)),
            pl.BlockSpec((_HIDDEN, _BJ),
                         lambda j: (0, jnp.minimum(j, _NJ - 1))),
            pl.BlockSpec((_HIDDEN, _BJ),
                         lambda j: (0, jnp.minimum(j, _NJ - 1))),
            pl.BlockSpec((_BJ, _HIDDEN),
                         lambda j: (jnp.maximum(j - 1, 0), 0)),
        ],
        out_specs=pl.BlockSpec((_TOKENS, _HIDDEN), lambda j: (0, 0)),
        out_shape=jax.ShapeDtypeStruct((_TOKENS, _HIDDEN), jnp.float32),
        scratch_shapes=[pltpu.VMEM((_TOKENS, _BJ), jnp.float32)],
    )(x, W_gate, W_up, W_down)


# speedup vs baseline: 1.2717x; 1.0415x over previous
import jax
import jax.numpy as jnp
from jax.experimental import pallas as pl

_HIDDEN = 2048
_INTER = 2048
_TOKENS = 32
_BJ = 256


def _mlp_kernel(x_ref, wg_ref, wu_ref, wd_ref, o_ref):
    j = pl.program_id(0)
    x = x_ref[...]
    gate = jnp.dot(x, wg_ref[...], preferred_element_type=jnp.float32)
    up = jnp.dot(x, wu_ref[...], preferred_element_type=jnp.float32)
    act = gate * jax.nn.sigmoid(gate) * up
    contrib = jnp.dot(act, wd_ref[...], preferred_element_type=jnp.float32)

    @pl.when(j == 0)
    def _init():
        o_ref[...] = contrib

    @pl.when(j > 0)
    def _acc():
        o_ref[...] += contrib


def kernel(x, W_gate, W_up, W_down):
    return pl.pallas_call(
        _mlp_kernel,
        grid=(_INTER // _BJ,),
        in_specs=[
            pl.BlockSpec((_TOKENS, _HIDDEN), lambda j: (0, 0)),
            pl.BlockSpec((_HIDDEN, _BJ), lambda j: (0, j)),
            pl.BlockSpec((_HIDDEN, _BJ), lambda j: (0, j)),
            pl.BlockSpec((_BJ, _HIDDEN), lambda j: (j, 0)),
        ],
        out_specs=pl.BlockSpec((_TOKENS, _HIDDEN), lambda j: (0, 0)),
        out_shape=jax.ShapeDtypeStruct((_TOKENS, _HIDDEN), jnp.float32),
    )(x, W_gate, W_up, W_down)
